# baseline (speedup 1.0000x reference)
import functools

import jax
import jax.numpy as jnp
from jax import lax
from jax.experimental import pallas as pl
from jax.experimental.pallas import tpu as pltpu
from jax.experimental.pallas import tpu_sc as plsc

_PREC = lax.Precision.HIGHEST
_NC = 2
_NS = 16
_CH = 80


def _seg_pad(n):
    rows_per_sub = ((n + _NS - 1) // _NS + 7) // 8 * 8
    return rows_per_sub, rows_per_sub * _NS


def _edge_segment_sum(h, src3, dst3, zeros):
    n, d = h.shape
    nw, steps, ch = src3.shape
    half = steps // 2
    rows_per_sub, npad = _seg_pad(n)
    mesh = plsc.VectorSubcoreMesh(core_axis_name="c", subcore_axis_name="s")

    @functools.partial(
        pl.kernel,
        out_type=jax.ShapeDtypeStruct((_NC, npad, d), jnp.float32),
        mesh=mesh,
        scratch_types=[
            pltpu.VMEM((half, ch), jnp.int32),
            pltpu.VMEM((half, ch), jnp.int32),
            pltpu.VMEM((ch, d), jnp.float32),
            pltpu.VMEM((ch, d), jnp.float32),
            pltpu.VMEM_SHARED((npad, d), jnp.float32),
            pltpu.SemaphoreType.DMA,
            pltpu.SemaphoreType.DMA,
            pltpu.SemaphoreType.DMA,
            pltpu.SemaphoreType.DMA,
        ],
    )
    def k(h_hbm, src_hbm, dst_hbm, z_hbm, out_hbm,
          sidx, didx, rows0, rows1, agg, gs0, gs1, ss0, ss1):
        c = lax.axis_index("c")
        s = lax.axis_index("s")
        wid = s * _NC + c
        rbase = s * rows_per_sub
        zc = pltpu.async_copy(z_hbm.at[pl.ds(rbase, rows_per_sub)],
                              agg.at[pl.ds(rbase, rows_per_sub)], ss0)
        pltpu.sync_copy(src_hbm.at[wid].at[pl.ds(0, half)], sidx)
        pltpu.sync_copy(dst_hbm.at[wid].at[pl.ds(0, half)], didx)
        zc.wait()
        plsc.subcore_barrier()

        for ph in range(2):
            @pl.loop(0, half // 2)
            def _(j):
                i0 = 2 * j
                i1 = i0 + 1
                g0 = pltpu.async_copy(h_hbm.at[sidx.at[i0]], rows0, gs0)
                g1 = pltpu.async_copy(h_hbm.at[sidx.at[i1]], rows1, gs1)
                g0.wait()
                s0 = pltpu.async_copy(rows0, agg.at[didx.at[i0]], ss0,
                                      add=True)
                g1.wait()
                s1 = pltpu.async_copy(rows1, agg.at[didx.at[i1]], ss1,
                                      add=True)
                s0.wait()
                s1.wait()

            if ph == 0:
                pltpu.sync_copy(src_hbm.at[wid].at[pl.ds(half, half)], sidx)
                pltpu.sync_copy(dst_hbm.at[wid].at[pl.ds(half, half)], didx)

        plsc.subcore_barrier()
        pltpu.sync_copy(agg.at[pl.ds(rbase, rows_per_sub)],
                        out_hbm.at[c].at[pl.ds(rbase, rows_per_sub)])

    return k(h, src3, dst3, zeros)


def _pad_edges(src, dst, n):
    e = src.shape[0]
    nw = _NC * _NS
    ch = _CH
    per_w = -(-e // nw)
    steps = -(-per_w // ch)
    steps = -(-steps // 16) * 16
    epad = nw * steps * ch
    _, npad = _seg_pad(n)
    trash = n + jnp.arange(epad - e, dtype=jnp.int32) % (npad - n)
    src_p = jnp.concatenate(
        [src, jnp.zeros((epad - e,), jnp.int32)]).reshape(nw, steps, ch)
    dst_p = jnp.concatenate([dst, trash]).reshape(nw, steps, ch)
    return src_p, dst_p


def _gin_layer(h, parts, n2g_col, eps, w1, b1, g1, be1, w2, b2, gl, bel,
               pool_input):
    n, d = h.shape
    hh = w1.shape[1]
    ng = 64

    def body(*refs):
        if pool_input:
            (h_ref, p_ref, n2g_ref, w1_ref, b1_ref, g1_ref, be1_ref,
             w2_ref, b2_ref, gl_ref, bel_ref, eps_ref,
             ho_ref, po_ref, pi_ref) = refs
        else:
            (h_ref, p_ref, n2g_ref, w1_ref, b1_ref, g1_ref, be1_ref,
             w2_ref, b2_ref, gl_ref, bel_ref, eps_ref,
             ho_ref, po_ref) = refs
        hcur = h_ref[...]
        n_rows = hcur.shape[0]
        z = (1.0 + eps_ref[0]) * hcur + p_ref[0, :n_rows] + p_ref[1, :n_rows]
        t = jnp.dot(z, w1_ref[...], precision=_PREC) + b1_ref[...]
        m = jnp.mean(t, axis=0, keepdims=True)
        v = jnp.mean((t - m) ** 2, axis=0, keepdims=True)
        u = jnp.maximum(
            g1_ref[...] * (t - m) / jnp.sqrt(v + 1e-5) + be1_ref[...], 0.0)
        t2 = jnp.dot(u, w2_ref[...], precision=_PREC) + b2_ref[...]
        m2 = jnp.mean(t2, axis=0, keepdims=True)
        v2 = jnp.mean((t2 - m2) ** 2, axis=0, keepdims=True)
        hn = jnp.maximum(
            gl_ref[...] * (t2 - m2) / jnp.sqrt(v2 + 1e-5) + bel_ref[...], 0.0)
        ho_ref[...] = hn
        onehot = (n2g_ref[...] ==
                  lax.broadcasted_iota(jnp.int32, (n, ng), 1)).astype(jnp.float32)
        dn = (((0,), (0,)), ((), ()))
        po_ref[...] = lax.dot_general(onehot, hn, dn, precision=_PREC)
        if pool_input:
            pi_ref[...] = lax.dot_general(onehot, hcur, dn, precision=_PREC)

    out_shapes = [jax.ShapeDtypeStruct((n, hh), jnp.float32),
                  jax.ShapeDtypeStruct((ng, hh), jnp.float32)]
    if pool_input:
        out_shapes.append(jax.ShapeDtypeStruct((ng, d), jnp.float32))
    in_specs = [pl.BlockSpec()] * 11 + [pl.BlockSpec(memory_space=pltpu.SMEM)]
    return pl.pallas_call(
        body,
        out_shape=out_shapes,
        in_specs=in_specs,
        out_specs=[pl.BlockSpec()] * len(out_shapes),
    )(h, parts, n2g_col, w1, b1, g1, be1, w2, b2, gl, bel, eps)


def _readout(pooled, wp, bp):
    ngr, nl, _, hh = pooled.shape
    o = wp.shape[2]

    def body(p_ref, w_ref, b_ref, o_ref):
        for g in range(ngr):
            acc = jnp.zeros((64, o), jnp.float32)
            for l in range(nl):
                acc = acc + jnp.dot(p_ref[g, l], w_ref[l], precision=_PREC)
                acc = acc + b_ref[l]
            nrm = jnp.sqrt(jnp.sum(acc * acc, axis=-1, keepdims=True))
            acc = acc / jnp.maximum(nrm, 1e-5)
            o_ref[:, g * o:(g + 1) * o] = acc

    return pl.pallas_call(
        body,
        out_shape=jax.ShapeDtypeStruct((64, ngr * o), jnp.float32),
    )(pooled, wp, bp)


def kernel(feat0, edge_index0, node2graph0, feat1, edge_index1, node2graph1,
           params):
    n, d = feat0.shape
    lps = [params['layer%d' % l] for l in range(3)]

    _, npad = _seg_pad(n)
    zeros = jnp.zeros((npad, d), jnp.float32)

    def run_graph(feat, edge_index, node2graph):
        src3, dst3 = _pad_edges(edge_index[0], edge_index[1], n)
        n2g_col = node2graph.reshape(n, 1)
        pooled = []
        h = feat
        for l, p in enumerate(lps):
            parts = _edge_segment_sum(h, src3, dst3, zeros)
            eps = jnp.reshape(p['eps'], (1,)).astype(jnp.float32)
            outs = _gin_layer(
                h, parts, n2g_col, eps,
                p['W1'], p['b1'].reshape(1, -1), p['g1'].reshape(1, -1),
                p['be1'].reshape(1, -1), p['W2'], p['b2'].reshape(1, -1),
                p['gL'].reshape(1, -1), p['beL'].reshape(1, -1),
                pool_input=(l == 0))
            if l == 0:
                h, pool_next, pool_in = outs
                pooled.append(pool_in)
            else:
                h, pool_next = outs
            pooled.append(pool_next)
        return jnp.stack(pooled)

    pooled0 = run_graph(feat0, edge_index0, node2graph0)
    pooled1 = run_graph(feat1, edge_index1, node2graph1)
    pooled = jnp.stack([pooled0, pooled1])
    wp = jnp.stack([params['pred%d' % l]['W'] for l in range(4)])
    bp = jnp.stack([params['pred%d' % l]['b'].reshape(1, -1)
                    for l in range(4)])
    return _readout(pooled, wp, bp)


# speedup vs baseline: 1.7207x; 1.7207x over previous
import functools

import jax
import jax.numpy as jnp
from jax import lax
from jax.experimental import pallas as pl
from jax.experimental.pallas import tpu as pltpu
from jax.experimental.pallas import tpu_sc as plsc

_PREC = lax.Precision.HIGHEST
_NC = 2
_NS = 16
_CH = 80


def _seg_pad(n):
    rows_per_sub = ((n + _NS - 1) // _NS + 7) // 8 * 8
    return rows_per_sub, rows_per_sub * _NS


def _edge_segment_sum(h, src3, dst3, zeros):
    n, d = h.shape
    e = src3.shape[0]
    nw = _NC * _NS
    ch = _CH
    per_w = e // nw
    steps = per_w // ch
    rows_per_sub, npad = _seg_pad(n)
    mesh = plsc.VectorSubcoreMesh(core_axis_name="c", subcore_axis_name="s")

    @functools.partial(
        pl.kernel,
        out_type=jax.ShapeDtypeStruct((_NC, npad, d), jnp.float32),
        mesh=mesh,
        scratch_types=[
            pltpu.VMEM((ch,), jnp.int32),
            pltpu.VMEM((ch,), jnp.int32),
            pltpu.VMEM((ch, d), jnp.float32),
            pltpu.VMEM((ch, d), jnp.float32),
            pltpu.VMEM_SHARED((npad, d), jnp.float32),
            pltpu.SemaphoreType.DMA,
            pltpu.SemaphoreType.DMA,
            pltpu.SemaphoreType.DMA,
            pltpu.SemaphoreType.DMA,
        ],
    )
    def k(h_hbm, src_hbm, dst_hbm, z_hbm, out_hbm,
          sidx, didx, rows0, rows1, agg, gs0, gs1, ss0, ss1):
        c = lax.axis_index("c")
        s = lax.axis_index("s")
        wid = s * _NC + c
        base = wid * per_w
        rbase = s * rows_per_sub
        pltpu.sync_copy(z_hbm.at[pl.ds(rbase, rows_per_sub)],
                        agg.at[pl.ds(rbase, rows_per_sub)])
        plsc.subcore_barrier()

        @pl.loop(0, steps)
        def _(i):
            off = base + i * ch
            pltpu.sync_copy(src_hbm.at[pl.ds(off, ch)], sidx)
            pltpu.sync_copy(dst_hbm.at[pl.ds(off, ch)], didx)
            pltpu.async_copy(h_hbm.at[sidx], rows0, gs0).wait()
            pltpu.sync_copy(rows0, agg.at[didx], add=True)

        plsc.subcore_barrier()
        pltpu.sync_copy(agg.at[pl.ds(rbase, rows_per_sub)],
                        out_hbm.at[c].at[pl.ds(rbase, rows_per_sub)])

    return k(h, src3, dst3, zeros)


def _pad_edges(src, dst, n):
    e = src.shape[0]
    nw = _NC * _NS
    ch = _CH
    per_w = -(-e // nw)
    steps = -(-per_w // ch)
    steps = -(-steps // 16) * 16
    epad = nw * steps * ch
    _, npad = _seg_pad(n)
    trash = n + jnp.arange(epad - e, dtype=jnp.int32) % (npad - n)
    src_p = jnp.concatenate(
        [src, jnp.zeros((epad - e,), jnp.int32)]).reshape(nw, steps, ch)
    dst_p = jnp.concatenate([dst, trash]).reshape(nw, steps, ch)
    return src_p, dst_p


def _gin_layer(h, parts, n2g_col, eps, w1, b1, g1, be1, w2, b2, gl, bel,
               pool_input):
    n, d = h.shape
    hh = w1.shape[1]
    ng = 64

    def body(*refs):
        if pool_input:
            (h_ref, p_ref, n2g_ref, w1_ref, b1_ref, g1_ref, be1_ref,
             w2_ref, b2_ref, gl_ref, bel_ref, eps_ref,
             ho_ref, po_ref, pi_ref) = refs
        else:
            (h_ref, p_ref, n2g_ref, w1_ref, b1_ref, g1_ref, be1_ref,
             w2_ref, b2_ref, gl_ref, bel_ref, eps_ref,
             ho_ref, po_ref) = refs
        hcur = h_ref[...]
        n_rows = hcur.shape[0]
        z = (1.0 + eps_ref[0]) * hcur + p_ref[0, :n_rows] + p_ref[1, :n_rows]
        t = jnp.dot(z, w1_ref[...], precision=_PREC) + b1_ref[...]
        m = jnp.mean(t, axis=0, keepdims=True)
        v = jnp.mean((t - m) ** 2, axis=0, keepdims=True)
        u = jnp.maximum(
            g1_ref[...] * (t - m) / jnp.sqrt(v + 1e-5) + be1_ref[...], 0.0)
        t2 = jnp.dot(u, w2_ref[...], precision=_PREC) + b2_ref[...]
        m2 = jnp.mean(t2, axis=0, keepdims=True)
        v2 = jnp.mean((t2 - m2) ** 2, axis=0, keepdims=True)
        hn = jnp.maximum(
            gl_ref[...] * (t2 - m2) / jnp.sqrt(v2 + 1e-5) + bel_ref[...], 0.0)
        ho_ref[...] = hn
        onehot = (n2g_ref[...] ==
                  lax.broadcasted_iota(jnp.int32, (n, ng), 1)).astype(jnp.float32)
        dn = (((0,), (0,)), ((), ()))
        po_ref[...] = lax.dot_general(onehot, hn, dn, precision=_PREC)
        if pool_input:
            pi_ref[...] = lax.dot_general(onehot, hcur, dn, precision=_PREC)

    out_shapes = [jax.ShapeDtypeStruct((n, hh), jnp.float32),
                  jax.ShapeDtypeStruct((ng, hh), jnp.float32)]
    if pool_input:
        out_shapes.append(jax.ShapeDtypeStruct((ng, d), jnp.float32))
    in_specs = [pl.BlockSpec()] * 11 + [pl.BlockSpec(memory_space=pltpu.SMEM)]
    return pl.pallas_call(
        body,
        out_shape=out_shapes,
        in_specs=in_specs,
        out_specs=[pl.BlockSpec()] * len(out_shapes),
    )(h, parts, n2g_col, w1, b1, g1, be1, w2, b2, gl, bel, eps)


def _readout(pooled, wp, bp):
    ngr, nl, _, hh = pooled.shape
    o = wp.shape[2]

    def body(p_ref, w_ref, b_ref, o_ref):
        for g in range(ngr):
            acc = jnp.zeros((64, o), jnp.float32)
            for l in range(nl):
                acc = acc + jnp.dot(p_ref[g, l], w_ref[l], precision=_PREC)
                acc = acc + b_ref[l]
            nrm = jnp.sqrt(jnp.sum(acc * acc, axis=-1, keepdims=True))
            acc = acc / jnp.maximum(nrm, 1e-5)
            o_ref[:, g * o:(g + 1) * o] = acc

    return pl.pallas_call(
        body,
        out_shape=jax.ShapeDtypeStruct((64, ngr * o), jnp.float32),
    )(pooled, wp, bp)


def kernel(feat0, edge_index0, node2graph0, feat1, edge_index1, node2graph1,
           params):
    n, d = feat0.shape
    lps = [params['layer%d' % l] for l in range(3)]

    _, npad = _seg_pad(n)
    zeros = jnp.zeros((npad, d), jnp.float32)

    def run_graph(feat, edge_index, node2graph):
        src3, dst3 = edge_index[0], edge_index[1]
        n2g_col = node2graph.reshape(n, 1)
        pooled = []
        h = feat
        for l, p in enumerate(lps):
            parts = _edge_segment_sum(h, src3, dst3, zeros)
            eps = jnp.reshape(p['eps'], (1,)).astype(jnp.float32)
            outs = _gin_layer(
                h, parts, n2g_col, eps,
                p['W1'], p['b1'].reshape(1, -1), p['g1'].reshape(1, -1),
                p['be1'].reshape(1, -1), p['W2'], p['b2'].reshape(1, -1),
                p['gL'].reshape(1, -1), p['beL'].reshape(1, -1),
                pool_input=(l == 0))
            if l == 0:
                h, pool_next, pool_in = outs
                pooled.append(pool_in)
            else:
                h, pool_next = outs
            pooled.append(pool_next)
        return jnp.stack(pooled)

    pooled0 = run_graph(feat0, edge_index0, node2graph0)
    pooled1 = run_graph(feat1, edge_index1, node2graph1)
    pooled = jnp.stack([pooled0, pooled1])
    wp = jnp.stack([params['pred%d' % l]['W'] for l in range(4)])
    bp = jnp.stack([params['pred%d' % l]['b'].reshape(1, -1)
                    for l in range(4)])
    return _readout(pooled, wp, bp)


# speedup vs baseline: 2.0012x; 1.1630x over previous
import functools

import jax
import jax.numpy as jnp
from jax import lax
from jax.experimental import pallas as pl
from jax.experimental.pallas import tpu as pltpu
from jax.experimental.pallas import tpu_sc as plsc

_PREC = lax.Precision.HIGHEST
_NC = 2
_NS = 16
_CH = 80


def _seg_pad(n):
    rows_per_sub = ((n + _NS - 1) // _NS + 7) // 8 * 8
    return rows_per_sub, rows_per_sub * _NS


def _edge_segment_sum(h, src3, dst3, zeros):
    n, d = h.shape
    e = src3.shape[0]
    nw = _NC * _NS
    ch = _CH
    per_w = e // nw
    steps = per_w // ch
    rows_per_sub, npad = _seg_pad(n)
    mesh = plsc.VectorSubcoreMesh(core_axis_name="c", subcore_axis_name="s")

    @functools.partial(
        pl.kernel,
        out_type=jax.ShapeDtypeStruct((_NC, npad, d), jnp.float32),
        mesh=mesh,
        scratch_types=[
            pltpu.VMEM((ch,), jnp.int32),
            pltpu.VMEM((ch,), jnp.int32),
            pltpu.VMEM((ch,), jnp.int32),
            pltpu.VMEM((ch,), jnp.int32),
            pltpu.VMEM((ch, d), jnp.float32),
            pltpu.VMEM((ch, d), jnp.float32),
            pltpu.VMEM_SHARED((npad, d), jnp.float32),
            pltpu.SemaphoreType.DMA,
            pltpu.SemaphoreType.DMA,
            pltpu.SemaphoreType.DMA,
            pltpu.SemaphoreType.DMA,
        ],
    )
    def k(h_hbm, src_hbm, dst_hbm, z_hbm, out_hbm,
          sidx0, didx0, sidx1, didx1, rows0, rows1, agg, gs0, gs1, ss0, ss1):
        c = lax.axis_index("c")
        s = lax.axis_index("s")
        wid = s * _NC + c
        base = wid * per_w
        rbase = s * rows_per_sub
        pltpu.sync_copy(z_hbm.at[pl.ds(rbase, rows_per_sub)],
                        agg.at[pl.ds(rbase, rows_per_sub)])
        plsc.subcore_barrier()

        @pl.loop(0, steps // 2)
        def _(j):
            off0 = base + (2 * j) * ch
            off1 = off0 + ch
            pltpu.sync_copy(src_hbm.at[pl.ds(off0, ch)], sidx0)
            pltpu.sync_copy(dst_hbm.at[pl.ds(off0, ch)], didx0)
            pltpu.sync_copy(src_hbm.at[pl.ds(off1, ch)], sidx1)
            pltpu.sync_copy(dst_hbm.at[pl.ds(off1, ch)], didx1)
            g0 = pltpu.async_copy(h_hbm.at[sidx0], rows0, gs0)
            g1 = pltpu.async_copy(h_hbm.at[sidx1], rows1, gs1)
            g0.wait()
            s0 = pltpu.async_copy(rows0, agg.at[didx0], ss0, add=True)
            g1.wait()
            s1 = pltpu.async_copy(rows1, agg.at[didx1], ss1, add=True)
            s0.wait()
            s1.wait()

        if steps % 2:
            off = base + (steps - 1) * ch
            pltpu.sync_copy(src_hbm.at[pl.ds(off, ch)], sidx0)
            pltpu.sync_copy(dst_hbm.at[pl.ds(off, ch)], didx0)
            pltpu.async_copy(h_hbm.at[sidx0], rows0, gs0).wait()
            pltpu.sync_copy(rows0, agg.at[didx0], add=True)

        plsc.subcore_barrier()
        pltpu.sync_copy(agg.at[pl.ds(rbase, rows_per_sub)],
                        out_hbm.at[c].at[pl.ds(rbase, rows_per_sub)])

    return k(h, src3, dst3, zeros)


def _pad_edges(src, dst, n):
    e = src.shape[0]
    nw = _NC * _NS
    ch = _CH
    per_w = -(-e // nw)
    steps = -(-per_w // ch)
    steps = -(-steps // 16) * 16
    epad = nw * steps * ch
    _, npad = _seg_pad(n)
    trash = n + jnp.arange(epad - e, dtype=jnp.int32) % (npad - n)
    src_p = jnp.concatenate(
        [src, jnp.zeros((epad - e,), jnp.int32)]).reshape(nw, steps, ch)
    dst_p = jnp.concatenate([dst, trash]).reshape(nw, steps, ch)
    return src_p, dst_p


def _gin_layer(h, parts, n2g_col, eps, w1, b1, g1, be1, w2, b2, gl, bel,
               pool_input):
    n, d = h.shape
    hh = w1.shape[1]
    ng = 64

    def body(*refs):
        if pool_input:
            (h_ref, p_ref, n2g_ref, w1_ref, b1_ref, g1_ref, be1_ref,
             w2_ref, b2_ref, gl_ref, bel_ref, eps_ref,
             ho_ref, po_ref, pi_ref) = refs
        else:
            (h_ref, p_ref, n2g_ref, w1_ref, b1_ref, g1_ref, be1_ref,
             w2_ref, b2_ref, gl_ref, bel_ref, eps_ref,
             ho_ref, po_ref) = refs
        hcur = h_ref[...]
        n_rows = hcur.shape[0]
        z = (1.0 + eps_ref[0]) * hcur + p_ref[0, :n_rows] + p_ref[1, :n_rows]
        t = jnp.dot(z, w1_ref[...], precision=_PREC) + b1_ref[...]
        m = jnp.mean(t, axis=0, keepdims=True)
        v = jnp.mean((t - m) ** 2, axis=0, keepdims=True)
        u = jnp.maximum(
            g1_ref[...] * (t - m) / jnp.sqrt(v + 1e-5) + be1_ref[...], 0.0)
        t2 = jnp.dot(u, w2_ref[...], precision=_PREC) + b2_ref[...]
        m2 = jnp.mean(t2, axis=0, keepdims=True)
        v2 = jnp.mean((t2 - m2) ** 2, axis=0, keepdims=True)
        hn = jnp.maximum(
            gl_ref[...] * (t2 - m2) / jnp.sqrt(v2 + 1e-5) + bel_ref[...], 0.0)
        ho_ref[...] = hn
        onehot = (n2g_ref[...] ==
                  lax.broadcasted_iota(jnp.int32, (n, ng), 1)).astype(jnp.float32)
        dn = (((0,), (0,)), ((), ()))
        po_ref[...] = lax.dot_general(onehot, hn, dn, precision=_PREC)
        if pool_input:
            pi_ref[...] = lax.dot_general(onehot, hcur, dn, precision=_PREC)

    out_shapes = [jax.ShapeDtypeStruct((n, hh), jnp.float32),
                  jax.ShapeDtypeStruct((ng, hh), jnp.float32)]
    if pool_input:
        out_shapes.append(jax.ShapeDtypeStruct((ng, d), jnp.float32))
    in_specs = [pl.BlockSpec()] * 11 + [pl.BlockSpec(memory_space=pltpu.SMEM)]
    return pl.pallas_call(
        body,
        out_shape=out_shapes,
        in_specs=in_specs,
        out_specs=[pl.BlockSpec()] * len(out_shapes),
    )(h, parts, n2g_col, w1, b1, g1, be1, w2, b2, gl, bel, eps)


def _readout(pooled, wp, bp):
    ngr, nl, _, hh = pooled.shape
    o = wp.shape[2]

    def body(p_ref, w_ref, b_ref, o_ref):
        for g in range(ngr):
            acc = jnp.zeros((64, o), jnp.float32)
            for l in range(nl):
                acc = acc + jnp.dot(p_ref[g, l], w_ref[l], precision=_PREC)
                acc = acc + b_ref[l]
            nrm = jnp.sqrt(jnp.sum(acc * acc, axis=-1, keepdims=True))
            acc = acc / jnp.maximum(nrm, 1e-5)
            o_ref[:, g * o:(g + 1) * o] = acc

    return pl.pallas_call(
        body,
        out_shape=jax.ShapeDtypeStruct((64, ngr * o), jnp.float32),
    )(pooled, wp, bp)


def kernel(feat0, edge_index0, node2graph0, feat1, edge_index1, node2graph1,
           params):
    n, d = feat0.shape
    lps = [params['layer%d' % l] for l in range(3)]

    _, npad = _seg_pad(n)
    zeros = jnp.zeros((npad, d), jnp.float32)

    def run_graph(feat, edge_index, node2graph):
        src3, dst3 = edge_index[0], edge_index[1]
        n2g_col = node2graph.reshape(n, 1)
        pooled = []
        h = feat
        for l, p in enumerate(lps):
            parts = _edge_segment_sum(h, src3, dst3, zeros)
            eps = jnp.reshape(p['eps'], (1,)).astype(jnp.float32)
            outs = _gin_layer(
                h, parts, n2g_col, eps,
                p['W1'], p['b1'].reshape(1, -1), p['g1'].reshape(1, -1),
                p['be1'].reshape(1, -1), p['W2'], p['b2'].reshape(1, -1),
                p['gL'].reshape(1, -1), p['beL'].reshape(1, -1),
                pool_input=(l == 0))
            if l == 0:
                h, pool_next, pool_in = outs
                pooled.append(pool_in)
            else:
                h, pool_next = outs
            pooled.append(pool_next)
        return jnp.stack(pooled)

    pooled0 = run_graph(feat0, edge_index0, node2graph0)
    pooled1 = run_graph(feat1, edge_index1, node2graph1)
    pooled = jnp.stack([pooled0, pooled1])
    wp = jnp.stack([params['pred%d' % l]['W'] for l in range(4)])
    bp = jnp.stack([params['pred%d' % l]['b'].reshape(1, -1)
                    for l in range(4)])
    return _readout(pooled, wp, bp)


# speedup vs baseline: 3.2501x; 1.6241x over previous
import functools

import jax
import jax.numpy as jnp
from jax import lax
from jax.experimental import pallas as pl
from jax.experimental.pallas import tpu as pltpu
from jax.experimental.pallas import tpu_sc as plsc

_PREC = lax.Precision.HIGHEST
_NC = 2
_NS = 16
_CH = 80


def _seg_pad(n):
    rows_per_sub = ((n + _NS - 1) // _NS + 7) // 8 * 8
    return rows_per_sub, rows_per_sub * _NS


def _edge_segment_sum(h, src3, dst3, zeros):
    n, d = h.shape
    e = src3.shape[0]
    nw = _NC * _NS
    ch = _CH
    per_w = e // nw
    steps = per_w // ch
    rows_per_sub, npad = _seg_pad(n)
    mesh = plsc.VectorSubcoreMesh(core_axis_name="c", subcore_axis_name="s")

    nblk = steps // 4
    tail = steps - 4 * nblk
    last = steps - 1

    @functools.partial(
        pl.kernel,
        out_type=jax.ShapeDtypeStruct((_NC, npad, d), jnp.float32),
        mesh=mesh,
        scratch_types=[
            pltpu.VMEM((ch,), jnp.int32),
            pltpu.VMEM((ch,), jnp.int32),
            pltpu.VMEM((ch,), jnp.int32),
            pltpu.VMEM((ch,), jnp.int32),
            pltpu.VMEM((ch,), jnp.int32),
            pltpu.VMEM((ch,), jnp.int32),
            pltpu.VMEM((ch, d), jnp.float32),
            pltpu.VMEM((ch, d), jnp.float32),
            pltpu.VMEM_SHARED((npad, d), jnp.float32),
            [pltpu.SemaphoreType.DMA] * 8,
        ],
    )
    def k(h_hbm, src_hbm, dst_hbm, z_hbm, out_hbm,
          sidx0, sidx1, didx0, didx1, didx2, didx3, rows0, rows1, agg, sems):
        didx = [didx0, didx1, didx2, didx3]
        gs0, gs1, ss0, ss1, is0, is1, id0, id1 = sems
        c = lax.axis_index("c")
        s = lax.axis_index("s")
        wid = s * _NC + c
        base = wid * per_w
        rbase = s * rows_per_sub

        def soff(i):
            return base + jnp.minimum(i, last) * ch

        def idx_cp(i, sbuf, dbuf, ssem, dsem):
            a = pltpu.make_async_copy(src_hbm.at[pl.ds(soff(i), ch)], sbuf,
                                      ssem)
            b = pltpu.make_async_copy(dst_hbm.at[pl.ds(soff(i), ch)], dbuf,
                                      dsem)
            return a, b

        def gath(i, sbuf, rbuf, sem):
            return pltpu.make_async_copy(h_hbm.at[sbuf], rbuf, sem)

        def scat(rbuf, dbuf, sem):
            return pltpu.make_async_copy(rbuf, agg.at[dbuf], sem)

        pltpu.sync_copy(z_hbm.at[pl.ds(rbase, rows_per_sub)],
                        agg.at[pl.ds(rbase, rows_per_sub)])
        plsc.subcore_barrier()

        def block(j, first):
            c0 = 4 * j
            if first:
                for cp in idx_cp(c0, sidx0, didx[0], is0, id0):
                    cp.start()
                for cp in idx_cp(c0 + 1, sidx1, didx[1], is1, id1):
                    cp.start()
            else:
                scat(rows0, didx[2], ss0).wait()
                scat(rows1, didx[3], ss1).wait()
            a, b = idx_cp(c0, sidx0, didx[0], is0, id0)
            a.wait()
            b.wait()
            g0 = gath(c0, sidx0, rows0, gs0)
            g0.start()
            a, b = idx_cp(c0 + 1, sidx1, didx[1], is1, id1)
            a.wait()
            b.wait()
            g1 = gath(c0 + 1, sidx1, rows1, gs1)
            g1.start()
            g0.wait()
            scat(rows0, didx[0], ss0).start(add=True)
            for cp in idx_cp(c0 + 2, sidx0, didx[2], is0, id0):
                cp.start()
            g1.wait()
            scat(rows1, didx[1], ss1).start(add=True)
            for cp in idx_cp(c0 + 3, sidx1, didx[3], is1, id1):
                cp.start()
            scat(rows0, didx[0], ss0).wait()
            a, b = idx_cp(c0 + 2, sidx0, didx[2], is0, id0)
            a.wait()
            b.wait()
            g2 = gath(c0 + 2, sidx0, rows0, gs0)
            g2.start()
            scat(rows1, didx[1], ss1).wait()
            a, b = idx_cp(c0 + 3, sidx1, didx[3], is1, id1)
            a.wait()
            b.wait()
            g3 = gath(c0 + 3, sidx1, rows1, gs1)
            g3.start()
            g2.wait()
            scat(rows0, didx[2], ss0).start(add=True)
            for cp in idx_cp(c0 + 4, sidx0, didx[0], is0, id0):
                cp.start()
            g3.wait()
            scat(rows1, didx[3], ss1).start(add=True)
            for cp in idx_cp(c0 + 5, sidx1, didx[1], is1, id1):
                cp.start()

        block(0, True)

        @pl.loop(1, nblk)
        def _(j):
            block(j, False)

        scat(rows0, didx[2], ss0).wait()
        scat(rows1, didx[3], ss1).wait()
        a, b = idx_cp(0, sidx0, didx[0], is0, id0)
        a.wait()
        b.wait()
        a, b = idx_cp(0, sidx1, didx[1], is1, id1)
        a.wait()
        b.wait()

        for t in range(tail):
            i = 4 * nblk + t
            pltpu.sync_copy(src_hbm.at[pl.ds(soff(i), ch)], sidx0)
            pltpu.sync_copy(dst_hbm.at[pl.ds(soff(i), ch)], didx[0])
            pltpu.async_copy(h_hbm.at[sidx0], rows0, gs0).wait()
            pltpu.sync_copy(rows0, agg.at[didx[0]], add=True)

        plsc.subcore_barrier()
        pltpu.sync_copy(agg.at[pl.ds(rbase, rows_per_sub)],
                        out_hbm.at[c].at[pl.ds(rbase, rows_per_sub)])

    return k(h, src3, dst3, zeros)


def _pad_edges(src, dst, n):
    e = src.shape[0]
    nw = _NC * _NS
    ch = _CH
    per_w = -(-e // nw)
    steps = -(-per_w // ch)
    steps = -(-steps // 16) * 16
    epad = nw * steps * ch
    _, npad = _seg_pad(n)
    trash = n + jnp.arange(epad - e, dtype=jnp.int32) % (npad - n)
    src_p = jnp.concatenate(
        [src, jnp.zeros((epad - e,), jnp.int32)]).reshape(nw, steps, ch)
    dst_p = jnp.concatenate([dst, trash]).reshape(nw, steps, ch)
    return src_p, dst_p


def _gin_layer(h, parts, n2g_col, eps, w1, b1, g1, be1, w2, b2, gl, bel,
               pool_input):
    n, d = h.shape
    hh = w1.shape[1]
    ng = 64

    def body(*refs):
        if pool_input:
            (h_ref, p_ref, n2g_ref, w1_ref, b1_ref, g1_ref, be1_ref,
             w2_ref, b2_ref, gl_ref, bel_ref, eps_ref,
             ho_ref, po_ref, pi_ref) = refs
        else:
            (h_ref, p_ref, n2g_ref, w1_ref, b1_ref, g1_ref, be1_ref,
             w2_ref, b2_ref, gl_ref, bel_ref, eps_ref,
             ho_ref, po_ref) = refs
        hcur = h_ref[...]
        n_rows = hcur.shape[0]
        z = (1.0 + eps_ref[0]) * hcur + p_ref[0, :n_rows] + p_ref[1, :n_rows]
        t = jnp.dot(z, w1_ref[...], precision=_PREC) + b1_ref[...]
        m = jnp.mean(t, axis=0, keepdims=True)
        v = jnp.mean((t - m) ** 2, axis=0, keepdims=True)
        u = jnp.maximum(
            g1_ref[...] * (t - m) / jnp.sqrt(v + 1e-5) + be1_ref[...], 0.0)
        t2 = jnp.dot(u, w2_ref[...], precision=_PREC) + b2_ref[...]
        m2 = jnp.mean(t2, axis=0, keepdims=True)
        v2 = jnp.mean((t2 - m2) ** 2, axis=0, keepdims=True)
        hn = jnp.maximum(
            gl_ref[...] * (t2 - m2) / jnp.sqrt(v2 + 1e-5) + bel_ref[...], 0.0)
        ho_ref[...] = hn
        onehot = (n2g_ref[...] ==
                  lax.broadcasted_iota(jnp.int32, (n, ng), 1)).astype(jnp.float32)
        dn = (((0,), (0,)), ((), ()))
        po_ref[...] = lax.dot_general(onehot, hn, dn, precision=_PREC)
        if pool_input:
            pi_ref[...] = lax.dot_general(onehot, hcur, dn, precision=_PREC)

    out_shapes = [jax.ShapeDtypeStruct((n, hh), jnp.float32),
                  jax.ShapeDtypeStruct((ng, hh), jnp.float32)]
    if pool_input:
        out_shapes.append(jax.ShapeDtypeStruct((ng, d), jnp.float32))
    in_specs = [pl.BlockSpec()] * 11 + [pl.BlockSpec(memory_space=pltpu.SMEM)]
    return pl.pallas_call(
        body,
        out_shape=out_shapes,
        in_specs=in_specs,
        out_specs=[pl.BlockSpec()] * len(out_shapes),
    )(h, parts, n2g_col, w1, b1, g1, be1, w2, b2, gl, bel, eps)


def _readout(pooled, wp, bp):
    ngr, nl, _, hh = pooled.shape
    o = wp.shape[2]

    def body(p_ref, w_ref, b_ref, o_ref):
        for g in range(ngr):
            acc = jnp.zeros((64, o), jnp.float32)
            for l in range(nl):
                acc = acc + jnp.dot(p_ref[g, l], w_ref[l], precision=_PREC)
                acc = acc + b_ref[l]
            nrm = jnp.sqrt(jnp.sum(acc * acc, axis=-1, keepdims=True))
            acc = acc / jnp.maximum(nrm, 1e-5)
            o_ref[:, g * o:(g + 1) * o] = acc

    return pl.pallas_call(
        body,
        out_shape=jax.ShapeDtypeStruct((64, ngr * o), jnp.float32),
    )(pooled, wp, bp)


def kernel(feat0, edge_index0, node2graph0, feat1, edge_index1, node2graph1,
           params):
    n, d = feat0.shape
    lps = [params['layer%d' % l] for l in range(3)]

    _, npad = _seg_pad(n)
    zeros = jnp.zeros((npad, d), jnp.float32)

    def run_graph(feat, edge_index, node2graph):
        src3, dst3 = edge_index[0], edge_index[1]
        n2g_col = node2graph.reshape(n, 1)
        pooled = []
        h = feat
        for l, p in enumerate(lps):
            parts = _edge_segment_sum(h, src3, dst3, zeros)
            eps = jnp.reshape(p['eps'], (1,)).astype(jnp.float32)
            outs = _gin_layer(
                h, parts, n2g_col, eps,
                p['W1'], p['b1'].reshape(1, -1), p['g1'].reshape(1, -1),
                p['be1'].reshape(1, -1), p['W2'], p['b2'].reshape(1, -1),
                p['gL'].reshape(1, -1), p['beL'].reshape(1, -1),
                pool_input=(l == 0))
            if l == 0:
                h, pool_next, pool_in = outs
                pooled.append(pool_in)
            else:
                h, pool_next = outs
            pooled.append(pool_next)
        return jnp.stack(pooled)

    pooled0 = run_graph(feat0, edge_index0, node2graph0)
    pooled1 = run_graph(feat1, edge_index1, node2graph1)
    pooled = jnp.stack([pooled0, pooled1])
    wp = jnp.stack([params['pred%d' % l]['W'] for l in range(4)])
    bp = jnp.stack([params['pred%d' % l]['b'].reshape(1, -1)
                    for l in range(4)])
    return _readout(pooled, wp, bp)
